# baseline (speedup 1.0000x reference)
import functools

import jax
import jax.numpy as jnp
from jax import lax
from jax.experimental import pallas as pl
from jax.experimental.pallas import tpu as pltpu
from jax.experimental.pallas import tpu_sc as plsc

NC, NS = 2, 16
NW = NC * NS
E = 320000
B = 128
NBLK = E // B
NB0 = NBLK // NW
REM = NBLK - NB0 * NW
D = 128
DE = 16
DO = D + DE + D
TR = DO // 8
TRS = D // 8
TRE = DE // 8
L = 16

_mesh = plsc.VectorSubcoreMesh(
    core_axis_name="c", subcore_axis_name="s", num_cores=NC, num_subcores=NS
)


@functools.partial(
    pl.kernel,
    out_type=jax.ShapeDtypeStruct((TR, NBLK, 8, B), jnp.float32),
    mesh=_mesh,
    scratch_types=[
        pltpu.VMEM((B,), jnp.int32),
        pltpu.VMEM((B,), jnp.int32),
        pltpu.VMEM((B, D), jnp.float32),
        pltpu.VMEM((TR, 8, B), jnp.float32),
        pltpu.SemaphoreType.DMA,
    ],
    compiler_params=pltpu.CompilerParams(use_tc_tiling_on_sc=True,
                                         needs_layout_passes=False),
)
def _triplet_cat_sc(x_hbm, ett_hbm, src_hbm, dst_hbm, out_hbm,
                    sidx, didx, g, otb, sem):
    wid = lax.axis_index("s") * NC + lax.axis_index("c")
    nb = jnp.where(wid < REM, NB0 + 1, NB0)
    sb = NB0 * wid + jnp.minimum(wid, REM)
    iotav = lax.iota(jnp.int32, L)

    def transpose_part(tr0):
        def trbody(t, carry):
            for fr in range(8):
                colv = jnp.full((L,), 8 * t + fr, jnp.int32)
                for e0 in range(B // L):
                    v = plsc.load_gather(g, [e0 * L + iotav, colv])
                    otb[tr0 + t, fr, pl.ds(e0 * L, L)] = v
            return carry

        lax.fori_loop(0, TRS, trbody, 0)

    def do_block(c):
        base = c * B
        pltpu.sync_copy(src_hbm.at[pl.ds(base, B)], sidx)
        pltpu.sync_copy(dst_hbm.at[pl.ds(base, B)], didx)
        pltpu.async_copy(x_hbm.at[sidx], g, sem).wait()
        transpose_part(0)
        pltpu.async_copy(x_hbm.at[didx], g, sem).wait()
        transpose_part(TRS + TRE)
        pltpu.sync_copy(ett_hbm.at[:, c], otb.at[pl.ds(TRS, TRE)])
        pltpu.sync_copy(otb, out_hbm.at[:, c])

    lax.fori_loop(0, nb, lambda i, carry: (do_block(sb + i), carry)[1], 0)


def kernel(x, edge_emb, edge_index):
    src = edge_index[0].astype(jnp.int32)
    dst = edge_index[1].astype(jnp.int32)
    ett = edge_emb.T.reshape(TRE, 8, NBLK, B).transpose(0, 2, 1, 3)
    ot4 = _triplet_cat_sc(x, ett, src, dst)
    return ot4.transpose(1, 3, 0, 2).reshape(E, DO)


# speedup vs baseline: 1.5704x; 1.5704x over previous
import functools

import jax
import jax.numpy as jnp
from jax import lax
from jax.experimental import pallas as pl
from jax.experimental.pallas import tpu as pltpu
from jax.experimental.pallas import tpu_sc as plsc

NC, NS = 2, 16
NW = NC * NS
E = 320000
B = 128
NBLK = E // B
NB0 = NBLK // NW
REM = NBLK - NB0 * NW
D = 128
DE = 16
DO = D + DE + D
TR = DO // 8
TRS = D // 8
TRE = DE // 8
L = 16

_mesh = plsc.VectorSubcoreMesh(
    core_axis_name="c", subcore_axis_name="s", num_cores=NC, num_subcores=NS
)


@functools.partial(
    pl.kernel,
    out_type=jax.ShapeDtypeStruct((TR, NBLK, 8, B), jnp.float32),
    mesh=_mesh,
    scratch_types=[
        pltpu.VMEM((B,), jnp.int32),
        pltpu.VMEM((B,), jnp.int32),
        pltpu.VMEM((B, D), jnp.float32),
        pltpu.VMEM((TR, 8, B), jnp.float32),
        pltpu.SemaphoreType.DMA,
    ],
    compiler_params=pltpu.CompilerParams(use_tc_tiling_on_sc=True,
                                         needs_layout_passes=False),
)
def _triplet_cat_sc(x_hbm, ett_hbm, src_hbm, dst_hbm, out_hbm,
                    sidx, didx, g, otb, sem):
    wid = lax.axis_index("s") * NC + lax.axis_index("c")
    nb = jnp.where(wid < REM, NB0 + 1, NB0)
    sb = NB0 * wid + jnp.minimum(wid, REM)
    iotav = lax.iota(jnp.int32, L)

    def transpose_part(tr0):
        @plsc.parallel_loop(0, TRS)
        def trbody(t):
            for fr in range(8):
                colv = jnp.full((L,), 8 * t + fr, jnp.int32)
                for e0 in range(B // L):
                    v = plsc.load_gather(g, [e0 * L + iotav, colv])
                    otb[tr0 + t, fr, pl.ds(e0 * L, L)] = v

    def do_block(c):
        base = c * B
        pltpu.sync_copy(src_hbm.at[pl.ds(base, B)], sidx)
        pltpu.sync_copy(dst_hbm.at[pl.ds(base, B)], didx)
        pltpu.async_copy(x_hbm.at[sidx], g, sem).wait()
        transpose_part(0)
        pltpu.async_copy(x_hbm.at[didx], g, sem).wait()
        transpose_part(TRS + TRE)
        pltpu.sync_copy(ett_hbm.at[:, c], otb.at[pl.ds(TRS, TRE)])
        pltpu.sync_copy(otb, out_hbm.at[:, c])

    lax.fori_loop(0, nb, lambda i, carry: (do_block(sb + i), carry)[1], 0)


def kernel(x, edge_emb, edge_index):
    src = edge_index[0].astype(jnp.int32)
    dst = edge_index[1].astype(jnp.int32)
    ett = edge_emb.T.reshape(TRE, 8, NBLK, B).transpose(0, 2, 1, 3)
    ot4 = _triplet_cat_sc(x, ett, src, dst)
    return ot4.transpose(1, 3, 0, 2).reshape(E, DO)
